# baseline (speedup 1.0000x reference)
import jax
import jax.numpy as jnp
from jax import lax
from jax.experimental import pallas as pl
from jax.experimental.pallas import tpu as pltpu
from jax.experimental.pallas import tpu_sc as plsc

GRID = 90
R = GRID * GRID
D = 256
L = 16
NC, NS = 1, 16
NW = NC * NS

TC_BLK = 896
TC_NBLK = 9
TC_ROWS = TC_BLK * TC_NBLK

RPW = 32
NG = RPW // L
SC_BASE = TC_ROWS
LAST_START = ((R - RPW) // 8) * 8
TAIL = R - (LAST_START + RPW)
BIG_I = 2 ** 30


def _som_body(x_hbm, w_hbm, dist_out, idx_out, x_v, w_v, bd_v, bi_v):
    c = lax.axis_index("c")
    s = lax.axis_index("s")
    wid = s * NC + c
    start = jnp.minimum(SC_BASE + wid * RPW, LAST_START)
    start = pl.multiple_of(start, 32)

    pltpu.sync_copy(w_hbm.at[pl.ds(start, RPW)], w_v.at[pl.ds(0, RPW)])
    pltpu.sync_copy(x_hbm, x_v)

    @pl.when(wid == NW - 1)
    def _():
        pltpu.sync_copy(w_hbm.at[pl.ds(LAST_START + RPW, TAIL)],
                        w_v.at[pl.ds(RPW, TAIL)])

    lane = lax.iota(jnp.int32, L)
    rows = [lane + g * L for g in range(NG + 1)]
    last = wid == NW - 1

    def feat_step(j, accs):
        cols = (lane + j) & (D - 1)
        xj = plsc.load_gather(x_v, [cols])
        out = []
        for g in range(NG + 1):
            wv = plsc.load_gather(w_v, [rows[g], cols])
            dv = wv - xj
            out.append(accs[g] + dv * dv)
        return tuple(out)

    zero = jnp.zeros((L,), jnp.float32)
    accs = lax.fori_loop(0, D, feat_step, (zero,) * (NG + 1))

    best_d = jnp.full((L,), jnp.inf, jnp.float32)
    best_i = jnp.full((L,), BIG_I, jnp.int32)
    for g in range(NG + 1):
        gi = start.astype(jnp.int32) + rows[g]
        d = accs[g]
        better = ((d < best_d) | ((d == best_d) & (gi < best_i))) & (gi < R)
        if g == NG:
            better = better & last
        best_d = jnp.where(better, d, best_d)
        best_i = jnp.where(better, gi, best_i)

    bd_v[...] = best_d
    bi_v[...] = best_i
    pltpu.sync_copy(bd_v, dist_out.at[wid])
    pltpu.sync_copy(bi_v, idx_out.at[wid])


def _som_call(inputs, w):
    return pl.kernel(
        _som_body,
        mesh=plsc.VectorSubcoreMesh(core_axis_name="c", subcore_axis_name="s", num_cores=1),
        out_type=[
            jax.ShapeDtypeStruct((NW, L), jnp.float32),
            jax.ShapeDtypeStruct((NW, L), jnp.int32),
        ],
        scratch_types=[
            pltpu.VMEM((D,), jnp.float32),
            pltpu.VMEM((RPW + L, D), jnp.float32),
            pltpu.VMEM((L,), jnp.float32),
            pltpu.VMEM((L,), jnp.int32),
        ],
        compiler_params=pltpu.CompilerParams(needs_layout_passes=False),
    )(inputs, w)


def _tc_body(x_ref, w_ref, od_ref, oi_ref):
    i = pl.program_id(0)
    dv = w_ref[...] - x_ref[...]
    acc = jnp.sum(dv * dv, axis=1, keepdims=True)
    ridx = lax.broadcasted_iota(jnp.int32, (TC_BLK, 1), 0) + i * TC_BLK
    m = jnp.min(acc)
    best = jnp.min(jnp.where(acc == m, ridx, BIG_I))
    od_ref[i, 0] = m
    oi_ref[i, 0] = best


def _tc_call(inputs, w):
    return pl.pallas_call(
        _tc_body,
        grid=(TC_NBLK,),
        in_specs=[
            pl.BlockSpec((1, D), lambda i: (0, 0)),
            pl.BlockSpec((TC_BLK, D), lambda i: (i, 0)),
        ],
        out_shape=[
            jax.ShapeDtypeStruct((TC_NBLK, 1), jnp.float32),
            jax.ShapeDtypeStruct((TC_NBLK, 1), jnp.int32),
        ],
        out_specs=[
            pl.BlockSpec(memory_space=pltpu.SMEM),
            pl.BlockSpec(memory_space=pltpu.SMEM),
        ],
    )(inputs.reshape(1, D), w)


def _merge_body(d_ref, i_ref, td_ref, ti_ref, o_ref):
    d = d_ref[...]
    i = i_ref[...]
    td = td_ref[...]
    ti = ti_ref[...]
    m = jnp.minimum(jnp.min(d), jnp.min(td))
    best = jnp.minimum(jnp.min(jnp.where(d == m, i, BIG_I)),
                       jnp.min(jnp.where(td == m, ti, BIG_I)))
    o_ref[0] = best // GRID
    o_ref[1] = best - (best // GRID) * GRID


def kernel(inputs, w):
    dists, idxs = _som_call(inputs, w)
    tc_d, tc_i = _tc_call(inputs, w)
    out = pl.pallas_call(
        _merge_body,
        out_shape=jax.ShapeDtypeStruct((2,), jnp.int32),
        out_specs=pl.BlockSpec(memory_space=pltpu.SMEM),
    )(dists, idxs, tc_d, tc_i)
    return out.astype(jnp.int64)


# speedup vs baseline: 1.0456x; 1.0456x over previous
import jax
import jax.numpy as jnp
from jax import lax
from jax.experimental import pallas as pl
from jax.experimental.pallas import tpu as pltpu
from jax.experimental.pallas import tpu_sc as plsc

GRID = 90
R = GRID * GRID
D = 256
L = 16
NC, NS = 1, 16
NW = NC * NS

TC_BLK = 896
TC_NBLK = 8
TC_ROWS = TC_BLK * TC_NBLK

RPW = 64
NG = RPW // L
SC_BASE = TC_ROWS
LAST_START = ((R - RPW) // 8) * 8
TAIL = R - (LAST_START + RPW)
BIG_I = 2 ** 30


def _som_body(x_hbm, w_hbm, dist_out, idx_out, x_v, w_v, bd_v, bi_v, semw):
    c = lax.axis_index("c")
    s = lax.axis_index("s")
    wid = s * NC + c
    start = jnp.minimum(SC_BASE + wid * RPW, LAST_START)
    start = pl.multiple_of(start, 32)

    cpw = pltpu.make_async_copy(
        w_hbm.at[pl.ds(start, RPW)], w_v.at[pl.ds(0, RPW)], semw)
    cpw.start()
    pltpu.sync_copy(x_hbm, x_v)

    @pl.when(wid == NW - 1)
    def _():
        pltpu.sync_copy(w_hbm.at[pl.ds(LAST_START + RPW, TAIL)],
                        w_v.at[pl.ds(RPW, TAIL)])

    lane = lax.iota(jnp.int32, L)
    rows = [lane + g * L for g in range(NG + 1)]
    last = wid == NW - 1

    def feat_step(j, accs):
        cols = (lane + j) & (D - 1)
        xj = plsc.load_gather(x_v, [cols])
        out = []
        for g in range(NG + 1):
            wv = plsc.load_gather(w_v, [rows[g], cols])
            dv = wv - xj
            out.append(accs[g] + dv * dv)
        return tuple(out)

    cpw.wait()
    zero = jnp.zeros((L,), jnp.float32)
    accs = lax.fori_loop(0, D, feat_step, (zero,) * (NG + 1))

    best_d = jnp.full((L,), jnp.inf, jnp.float32)
    best_i = jnp.full((L,), BIG_I, jnp.int32)
    for g in range(NG + 1):
        gi = start.astype(jnp.int32) + rows[g]
        d = accs[g]
        better = ((d < best_d) | ((d == best_d) & (gi < best_i))) & (gi < R)
        if g == NG:
            better = better & last
        best_d = jnp.where(better, d, best_d)
        best_i = jnp.where(better, gi, best_i)

    bd_v[...] = best_d
    bi_v[...] = best_i
    pltpu.sync_copy(bd_v, dist_out.at[wid])
    pltpu.sync_copy(bi_v, idx_out.at[wid])


def _som_call(inputs, w):
    return pl.kernel(
        _som_body,
        mesh=plsc.VectorSubcoreMesh(core_axis_name="c", subcore_axis_name="s", num_cores=1),
        out_type=[
            jax.ShapeDtypeStruct((NW, L), jnp.float32),
            jax.ShapeDtypeStruct((NW, L), jnp.int32),
        ],
        scratch_types=[
            pltpu.VMEM((D,), jnp.float32),
            pltpu.VMEM((RPW + L, D), jnp.float32),
            pltpu.VMEM((L,), jnp.float32),
            pltpu.VMEM((L,), jnp.int32),
            pltpu.SemaphoreType.DMA,
        ],
        compiler_params=pltpu.CompilerParams(needs_layout_passes=False),
    )(inputs, w)


def _tc_body(x_ref, w_ref, od_ref, oi_ref):
    i = pl.program_id(0)
    dv = w_ref[...] - x_ref[...]
    acc = jnp.sum(dv * dv, axis=1, keepdims=True)
    ridx = lax.broadcasted_iota(jnp.int32, (TC_BLK, 1), 0) + i * TC_BLK
    m = jnp.min(acc)
    best = jnp.min(jnp.where(acc == m, ridx, BIG_I))
    od_ref[i, 0] = m
    oi_ref[i, 0] = best


def _tc_call(inputs, w):
    return pl.pallas_call(
        _tc_body,
        grid=(TC_NBLK,),
        in_specs=[
            pl.BlockSpec((1, D), lambda i: (0, 0)),
            pl.BlockSpec((TC_BLK, D), lambda i: (i, 0)),
        ],
        out_shape=[
            jax.ShapeDtypeStruct((TC_NBLK, 1), jnp.float32),
            jax.ShapeDtypeStruct((TC_NBLK, 1), jnp.int32),
        ],
        out_specs=[
            pl.BlockSpec(memory_space=pltpu.SMEM),
            pl.BlockSpec(memory_space=pltpu.SMEM),
        ],
    )(inputs.reshape(1, D), w)


def _merge_body(d_ref, i_ref, td_ref, ti_ref, o_ref):
    d = d_ref[...]
    i = i_ref[...]
    td = td_ref[...]
    ti = ti_ref[...]
    m = jnp.minimum(jnp.min(d), jnp.min(td))
    best = jnp.minimum(jnp.min(jnp.where(d == m, i, BIG_I)),
                       jnp.min(jnp.where(td == m, ti, BIG_I)))
    o_ref[0] = best // GRID
    o_ref[1] = best - (best // GRID) * GRID


def kernel(inputs, w):
    dists, idxs = _som_call(inputs, w)
    tc_d, tc_i = _tc_call(inputs, w)
    out = pl.pallas_call(
        _merge_body,
        out_shape=jax.ShapeDtypeStruct((2,), jnp.int32),
        out_specs=pl.BlockSpec(memory_space=pltpu.SMEM),
    )(dists, idxs, tc_d, tc_i)
    return out.astype(jnp.int64)


# speedup vs baseline: 1.0583x; 1.0121x over previous
import jax
import jax.numpy as jnp
from jax import lax
from jax.experimental import pallas as pl
from jax.experimental.pallas import tpu as pltpu
from jax.experimental.pallas import tpu_sc as plsc

GRID = 90
R = GRID * GRID
D = 256
L = 16
NC, NS = 1, 16
NW = NC * NS

TC_BLK = 896
TC_NBLK = 8
TC_ROWS = TC_BLK * TC_NBLK

RPW = 64
NG = RPW // L
SC_BASE = TC_ROWS
LAST_START = ((R - RPW) // 8) * 8
TAIL = R - (LAST_START + RPW)
BIG_I = 2 ** 30


def _som_body(x_hbm, w_hbm, dist_out, idx_out, x_v, w_v, bd_v, bi_v, semw):
    c = lax.axis_index("c")
    s = lax.axis_index("s")
    wid = s * NC + c
    start = jnp.minimum(SC_BASE + wid * RPW, LAST_START)
    start = pl.multiple_of(start, 32)

    cpw = pltpu.make_async_copy(
        w_hbm.at[pl.ds(start, RPW)], w_v.at[pl.ds(0, RPW)], semw)
    cpw.start()
    pltpu.sync_copy(x_hbm, x_v)

    @pl.when(wid == NW - 1)
    def _():
        pltpu.sync_copy(w_hbm.at[pl.ds(LAST_START + RPW, TAIL)],
                        w_v.at[pl.ds(RPW, TAIL)])

    lane = lax.iota(jnp.int32, L)
    rows = [lane + g * L for g in range(NG + 1)]
    last = wid == NW - 1

    def feat_step(j, accs):
        cols = (lane + j) & (D - 1)
        xj = plsc.load_gather(x_v, [cols])
        out = []
        for g in range(NG + 1):
            wv = plsc.load_gather(w_v, [rows[g], cols])
            dv = wv - xj
            out.append(accs[g] + dv * dv)
        return tuple(out)

    cpw.wait()
    zero = jnp.zeros((L,), jnp.float32)
    accs = lax.fori_loop(0, D, feat_step, (zero,) * (NG + 1))

    best_d = jnp.full((L,), jnp.inf, jnp.float32)
    best_i = jnp.full((L,), BIG_I, jnp.int32)
    for g in range(NG + 1):
        gi = start.astype(jnp.int32) + rows[g]
        d = accs[g]
        better = ((d < best_d) | ((d == best_d) & (gi < best_i))) & (gi < R)
        if g == NG:
            better = better & last
        best_d = jnp.where(better, d, best_d)
        best_i = jnp.where(better, gi, best_i)

    bd_v[...] = best_d
    bi_v[...] = best_i
    pltpu.sync_copy(bd_v, dist_out.at[wid])
    pltpu.sync_copy(bi_v, idx_out.at[wid])


def _som_call(inputs, w):
    return pl.kernel(
        _som_body,
        mesh=plsc.VectorSubcoreMesh(core_axis_name="c", subcore_axis_name="s", num_cores=1),
        out_type=[
            jax.ShapeDtypeStruct((NW, L), jnp.float32),
            jax.ShapeDtypeStruct((NW, L), jnp.int32),
        ],
        scratch_types=[
            pltpu.VMEM((D,), jnp.float32),
            pltpu.VMEM((RPW + L, D), jnp.float32),
            pltpu.VMEM((L,), jnp.float32),
            pltpu.VMEM((L,), jnp.int32),
            pltpu.SemaphoreType.DMA,
        ],
        compiler_params=pltpu.CompilerParams(needs_layout_passes=False, skip_device_barrier=True),
    )(inputs, w)


def _tc_body(x_ref, w_ref, od_ref, oi_ref):
    i = pl.program_id(0)
    dv = w_ref[...] - x_ref[...]
    acc = jnp.sum(dv * dv, axis=1, keepdims=True)
    ridx = lax.broadcasted_iota(jnp.int32, (TC_BLK, 1), 0) + i * TC_BLK
    m = jnp.min(acc)
    best = jnp.min(jnp.where(acc == m, ridx, BIG_I))
    od_ref[i, 0] = m
    oi_ref[i, 0] = best


def _tc_call(inputs, w):
    return pl.pallas_call(
        _tc_body,
        grid=(TC_NBLK,),
        in_specs=[
            pl.BlockSpec((1, D), lambda i: (0, 0)),
            pl.BlockSpec((TC_BLK, D), lambda i: (i, 0)),
        ],
        out_shape=[
            jax.ShapeDtypeStruct((TC_NBLK, 1), jnp.float32),
            jax.ShapeDtypeStruct((TC_NBLK, 1), jnp.int32),
        ],
        out_specs=[
            pl.BlockSpec(memory_space=pltpu.SMEM),
            pl.BlockSpec(memory_space=pltpu.SMEM),
        ],
    )(inputs.reshape(1, D), w)


def _merge_body(d_ref, i_ref, td_ref, ti_ref, o_ref):
    d = d_ref[...]
    i = i_ref[...]
    td = td_ref[...]
    ti = ti_ref[...]
    m = jnp.minimum(jnp.min(d), jnp.min(td))
    best = jnp.minimum(jnp.min(jnp.where(d == m, i, BIG_I)),
                       jnp.min(jnp.where(td == m, ti, BIG_I)))
    o_ref[0] = best // GRID
    o_ref[1] = best - (best // GRID) * GRID


def kernel(inputs, w):
    dists, idxs = _som_call(inputs, w)
    tc_d, tc_i = _tc_call(inputs, w)
    out = pl.pallas_call(
        _merge_body,
        out_shape=jax.ShapeDtypeStruct((2,), jnp.int32),
        out_specs=pl.BlockSpec(memory_space=pltpu.SMEM),
    )(dists, idxs, tc_d, tc_i)
    return out.astype(jnp.int64)
